# Initial kernel scaffold; baseline (speedup 1.0000x reference)
import jax
import jax.numpy as jnp
from jax.experimental import pallas as pl


def kernel(x, edge_index, attn_w, attn_b, fc_w, fc_b):
    raise NotImplementedError("write your pallas kernel here")


# speedup vs baseline: 4.1533x; 4.1533x over previous
import functools

import jax
import jax.numpy as jnp
from jax import lax
from jax.experimental import pallas as pl
from jax.experimental.pallas import tpu as pltpu
from jax.experimental.pallas import tpu_sc as plsc

N = 10000
E = 320000
D = 128
H = 4
HP = 16
NC = 2
NS = 16
NW = NC * NS
PER_W = E // NW
C = 80
NCHUNK = PER_W // C
NP = 10240
ROWS_T = NP // NS

_f32 = jnp.float32
_i32 = jnp.int32

_mesh = plsc.VectorSubcoreMesh(
    core_axis_name="c", subcore_axis_name="s", num_cores=NC, num_subcores=NS)

_sc_params = pltpu.CompilerParams(
    needs_layout_passes=False, use_tc_tiling_on_sc=False)


_RB = 400


def _dense_body(x_ref, fcw_ref, fcb_ref, w1_ref, w2_ref, ab_ref,
                fch_ref, as_ref, ad_ref):
    xb = x_ref[...]
    fch_ref[...] = jnp.dot(xb, fcw_ref[...],
                           preferred_element_type=_f32) + fcb_ref[...]
    as_ref[...] = jnp.dot(xb, w1_ref[...],
                          preferred_element_type=_f32) + ab_ref[...]
    ad_ref[...] = jnp.dot(xb, w2_ref[...],
                          preferred_element_type=_f32)


def _dense(x, fc_wT, fc_b2, w1T, w2T, attn_b2):
    grid = (N // _RB,)
    return pl.pallas_call(
        _dense_body,
        grid=grid,
        in_specs=[
            pl.BlockSpec((_RB, D), lambda i: (i, 0)),
            pl.BlockSpec((D, D), lambda i: (0, 0)),
            pl.BlockSpec((1, D), lambda i: (0, 0)),
            pl.BlockSpec((D, HP), lambda i: (0, 0)),
            pl.BlockSpec((D, HP), lambda i: (0, 0)),
            pl.BlockSpec((1, HP), lambda i: (0, 0)),
        ],
        out_specs=[
            pl.BlockSpec((_RB, D), lambda i: (i, 0)),
            pl.BlockSpec((_RB, HP), lambda i: (i, 0)),
            pl.BlockSpec((_RB, HP), lambda i: (i, 0)),
        ],
        out_shape=[
            jax.ShapeDtypeStruct((N, D), _f32),
            jax.ShapeDtypeStruct((N, HP), _f32),
            jax.ShapeDtypeStruct((N, HP), _f32),
        ],
    )(x, fc_wT, fc_b2, w1T, w2T, attn_b2)


@functools.partial(
    pl.kernel,
    compiler_params=_sc_params,
    out_type=(jax.ShapeDtypeStruct((E, H), _f32),
              jax.ShapeDtypeStruct((NC, NP, HP), _f32)),
    mesh=_mesh,
    scratch_types=[
        pltpu.VMEM((C,), _i32),
        pltpu.VMEM((C,), _i32),
        pltpu.VMEM((C, HP), _f32),
        pltpu.VMEM((C, HP), _f32),
        pltpu.VMEM((C, H), _f32),
        pltpu.VMEM((C, HP), _f32),
        pltpu.VMEM_SHARED((NP, HP), _f32),
        pltpu.SemaphoreType.DMA,
        pltpu.SemaphoreType.DMA,
    ],
)
def _pass1(a_src_hbm, a_dst_hbm, src_hbm, dst_hbm, zeros16_hbm,
           e_hbm, dp_hbm,
           srcv, dstv, asv, adv, ev4, ev16, dsh, sem1, sem2):
    c = lax.axis_index("c")
    s = lax.axis_index("s")
    base_w = (c * NS + s) * PER_W
    pltpu.sync_copy(zeros16_hbm.at[pl.ds(s * ROWS_T, ROWS_T)],
                    dsh.at[pl.ds(s * ROWS_T, ROWS_T)])

    zero16 = jnp.zeros((16,), _f32)

    def zb(j, carry):
        ev16[j, :] = zero16
        return carry

    lax.fori_loop(0, C, zb, 0)
    plsc.subcore_barrier()

    iota = lax.iota(_i32, 16)
    row_off = lax.shift_right_logical(iota, 2)
    col = lax.bitwise_and(iota, 3)

    def chunk(i, carry):
        base = base_w + i * C
        pltpu.sync_copy(src_hbm.at[pl.ds(base, C)], srcv)
        pltpu.sync_copy(dst_hbm.at[pl.ds(base, C)], dstv)
        pltpu.async_copy(a_src_hbm.at[srcv], asv, sem1).wait()
        pltpu.async_copy(a_dst_hbm.at[dstv], adv, sem2).wait()

        def vb(j, carry2):
            rows = j * (16 // H) + row_off
            t = plsc.load_gather(asv, [rows, col]) + plsc.load_gather(adv, [rows, col])
            t = jnp.where(t >= 0.0, t, 0.2 * t)
            plsc.store_scatter(ev4, [rows, col], t)
            plsc.store_scatter(ev16, [rows, col], t)
            return carry2

        lax.fori_loop(0, C * H // 16, vb, 0)
        pltpu.sync_copy(ev4, e_hbm.at[pl.ds(base, C)])
        pltpu.sync_copy(ev16, dsh.at[srcv], add=True)
        return carry

    lax.fori_loop(0, NCHUNK, chunk, 0)
    plsc.subcore_barrier()
    pltpu.sync_copy(dsh.at[pl.ds(s * ROWS_T, ROWS_T)],
                    dp_hbm.at[c, pl.ds(s * ROWS_T, ROWS_T)])


def _rdenom_body(dp_ref, rd_ref):
    rd_ref[...] = 1.0 / (dp_ref[0] + dp_ref[1])


def _rdenom(dp):
    dpr = dp.reshape(NC, NP * HP // 128, 128)
    out = pl.pallas_call(
        _rdenom_body,
        out_shape=jax.ShapeDtypeStruct((NP * HP // 128, 128), _f32),
    )(dpr)
    return out.reshape(NP, HP)


@functools.partial(
    pl.kernel,
    compiler_params=_sc_params,
    out_type=jax.ShapeDtypeStruct((NC, NP, D), _f32),
    mesh=_mesh,
    scratch_types=[
        pltpu.VMEM((C,), _i32),
        pltpu.VMEM((C,), _i32),
        pltpu.VMEM((C, H), _f32),
        pltpu.VMEM((C, HP), _f32),
        pltpu.VMEM((C,), _f32),
        pltpu.VMEM((C, D), _f32),
        pltpu.VMEM_SHARED((NP, D), _f32),
        pltpu.SemaphoreType.DMA,
        pltpu.SemaphoreType.DMA,
    ],
)
def _pass2(src_hbm, dst_hbm, e_hbm, rd_hbm, fch_hbm, zerosD_hbm,
           op_hbm,
           srcv, dstv, ev, rdv, coeffv, rows, osh, sem1, sem2):
    c = lax.axis_index("c")
    s = lax.axis_index("s")
    base_w = (c * NS + s) * PER_W
    pltpu.sync_copy(zerosD_hbm.at[pl.ds(s * ROWS_T, ROWS_T)],
                    osh.at[pl.ds(s * ROWS_T, ROWS_T)])
    plsc.subcore_barrier()

    iota = lax.iota(_i32, 16)

    def chunk(i, carry):
        base = base_w + i * C
        pltpu.sync_copy(src_hbm.at[pl.ds(base, C)], srcv)
        pltpu.sync_copy(dst_hbm.at[pl.ds(base, C)], dstv)
        pltpu.sync_copy(e_hbm.at[pl.ds(base, C)], ev)
        pltpu.async_copy(rd_hbm.at[srcv], rdv, sem1).wait()
        pltpu.async_copy(fch_hbm.at[dstv], rows, sem2).wait()

        def cb(j, carry2):
            ridx = j * 16 + iota
            acc = jnp.zeros((16,), _f32)
            for h in range(H):
                hidx = jnp.full((16,), h, _i32)
                acc = acc + (plsc.load_gather(ev, [ridx, hidx]) *
                             plsc.load_gather(rdv, [ridx, hidx]))
            coeffv[pl.ds(j * 16, 16)] = acc * (1.0 / H)
            return carry2

        lax.fori_loop(0, C // 16, cb, 0)

        def sb(j, carry2):
            cbv = plsc.load_gather(coeffv, [jnp.full((16,), j, _i32)])
            for k in range(D // 16):
                rows[j, pl.ds(k * 16, 16)] = rows[j, pl.ds(k * 16, 16)] * cbv
            return carry2

        lax.fori_loop(0, C, sb, 0)
        pltpu.sync_copy(rows, osh.at[srcv], add=True)
        return carry

    lax.fori_loop(0, NCHUNK, chunk, 0)
    plsc.subcore_barrier()
    pltpu.sync_copy(osh.at[pl.ds(s * ROWS_T, ROWS_T)],
                    op_hbm.at[c, pl.ds(s * ROWS_T, ROWS_T)])


def _final_body(op_ref, o_ref):
    o_ref[...] = jnp.maximum(op_ref[0] + op_ref[1], 0.0)


def _final(op):
    grid = (N // _RB,)
    return pl.pallas_call(
        _final_body,
        grid=grid,
        in_specs=[pl.BlockSpec((NC, _RB, D), lambda i: (0, i, 0))],
        out_specs=pl.BlockSpec((_RB, D), lambda i: (i, 0)),
        out_shape=jax.ShapeDtypeStruct((N, D), _f32),
    )(op)


def kernel(x, edge_index, attn_w, attn_b, fc_w, fc_b):
    src = edge_index[0].astype(_i32)
    dst = edge_index[1].astype(_i32)
    fc_wT = fc_w.T
    w1T = jnp.zeros((D, HP), _f32).at[:, :H].set(attn_w[:, :D].T)
    w2T = jnp.zeros((D, HP), _f32).at[:, :H].set(attn_w[:, D:].T)
    fc_b2 = fc_b.reshape(1, D)
    attn_b2 = jnp.zeros((1, HP), _f32).at[0, :H].set(attn_b)

    fc_h, a_src, a_dst = _dense(x, fc_wT, fc_b2, w1T, w2T, attn_b2)

    zeros16 = jnp.zeros((NP, HP), _f32)
    zerosD = jnp.zeros((NP, D), _f32)

    e, dp = _pass1(a_src, a_dst, src, dst, zeros16)
    rd = _rdenom(dp)
    op = _pass2(src, dst, e, rd, fc_h, zerosD)
    return _final(op[:, :N])


# speedup vs baseline: 10.4132x; 2.5072x over previous
import functools

import jax
import jax.numpy as jnp
from jax import lax
from jax.experimental import pallas as pl
from jax.experimental.pallas import tpu as pltpu
from jax.experimental.pallas import tpu_sc as plsc

N = 10000
E = 320000
D = 128
H = 4
HP = 16
NC = 2
NS = 16
NW = NC * NS
PER_W = E // NW
C = 80
NCHUNK = PER_W // C
NROW = E // C
C2 = 80
NCH2 = PER_W // C2
NROW2 = E // C2
NP = 10240
ROWS_T = NP // NS

_f32 = jnp.float32
_i32 = jnp.int32

_mesh = plsc.VectorSubcoreMesh(
    core_axis_name="c", subcore_axis_name="s", num_cores=NC, num_subcores=NS)

_sc_params = pltpu.CompilerParams(
    needs_layout_passes=False, use_tc_tiling_on_sc=False)


_RB = 400


def _dense_body(x_ref, fcw_ref, fcb_ref, w1_ref, w2_ref, ab_ref,
                fch_ref, as_ref, ad_ref):
    xb = x_ref[...]
    fch_ref[...] = jnp.dot(xb, fcw_ref[...],
                           preferred_element_type=_f32) + fcb_ref[...]
    as_ref[...] = jnp.dot(xb, w1_ref[...],
                          preferred_element_type=_f32) + ab_ref[...]
    ad_ref[...] = jnp.dot(xb, w2_ref[...],
                          preferred_element_type=_f32)


def _dense(x, fc_wT, fc_b2, w1T, w2T, attn_b2):
    grid = (N // _RB,)
    return pl.pallas_call(
        _dense_body,
        grid=grid,
        in_specs=[
            pl.BlockSpec((_RB, D), lambda i: (i, 0)),
            pl.BlockSpec((D, D), lambda i: (0, 0)),
            pl.BlockSpec((1, D), lambda i: (0, 0)),
            pl.BlockSpec((D, HP), lambda i: (0, 0)),
            pl.BlockSpec((D, HP), lambda i: (0, 0)),
            pl.BlockSpec((1, HP), lambda i: (0, 0)),
        ],
        out_specs=[
            pl.BlockSpec((_RB, D), lambda i: (i, 0)),
            pl.BlockSpec((_RB, HP), lambda i: (i, 0)),
            pl.BlockSpec((_RB, HP), lambda i: (i, 0)),
        ],
        out_shape=[
            jax.ShapeDtypeStruct((N, D), _f32),
            jax.ShapeDtypeStruct((N, HP), _f32),
            jax.ShapeDtypeStruct((N, HP), _f32),
        ],
    )(x, fc_wT, fc_b2, w1T, w2T, attn_b2)


@functools.partial(
    pl.kernel,
    compiler_params=_sc_params,
    out_type=(jax.ShapeDtypeStruct((E, H), _f32),
              jax.ShapeDtypeStruct((NC, NP, HP), _f32)),
    mesh=_mesh,
    scratch_types=[
        pltpu.VMEM((NCHUNK, C), _i32),
        pltpu.VMEM((NCHUNK, C), _i32),
        pltpu.VMEM((2, C, HP), _f32),
        pltpu.VMEM((2, C, HP), _f32),
        pltpu.VMEM((2, C, H), _f32),
        pltpu.VMEM((2, C, HP), _f32),
        pltpu.VMEM_SHARED((NP, HP), _f32),
        pltpu.SemaphoreType.DMA,
        pltpu.SemaphoreType.DMA,
        pltpu.SemaphoreType.DMA,
        pltpu.SemaphoreType.DMA,
    ],
)
def _pass1(a_src_hbm, a_dst_hbm, src2_hbm, dst2_hbm, zeros16_hbm,
           e_hbm, dp_hbm,
           srcall, dstall, asv, adv, ev4, ev16, dsh,
           gsem0, gsem1, esem0, esem1):
    c = lax.axis_index("c")
    s = lax.axis_index("s")
    w = c * NS + s
    base_w = w * PER_W
    pltpu.sync_copy(zeros16_hbm.at[pl.ds(s * ROWS_T, ROWS_T)],
                    dsh.at[pl.ds(s * ROWS_T, ROWS_T)])
    pltpu.sync_copy(src2_hbm.at[pl.ds(w * NCHUNK, NCHUNK)], srcall)
    pltpu.sync_copy(dst2_hbm.at[pl.ds(w * NCHUNK, NCHUNK)], dstall)

    zero16 = jnp.zeros((16,), _f32)

    def zb(j, carry):
        ev16[0, j, :] = zero16
        ev16[1, j, :] = zero16
        return carry

    lax.fori_loop(0, C, zb, 0)
    plsc.subcore_barrier()

    iota = lax.iota(_i32, 16)
    row_off = lax.shift_right_logical(iota, 2)
    col = lax.bitwise_and(iota, 3)

    def gissue(i, p, sem):
        pltpu.async_copy(a_src_hbm.at[srcall.at[i]], asv.at[p], sem)
        pltpu.async_copy(a_dst_hbm.at[dstall.at[i]], adv.at[p], sem)

    def gwait(i, p, sem):
        pltpu.make_async_copy(a_src_hbm.at[srcall.at[i]], asv.at[p], sem).wait()
        pltpu.make_async_copy(a_dst_hbm.at[dstall.at[i]], adv.at[p], sem).wait()

    gissue(0, 0, gsem0)

    def chunk(i, carry):
        p = lax.rem(i, 2)
        base = base_w + i * C

        @pl.when(p == 0)
        def _():
            gwait(i, 0, gsem0)

        @pl.when(p == 1)
        def _():
            gwait(i, 1, gsem1)

        @pl.when(jnp.logical_and(i + 1 < NCHUNK, p == 0))
        def _():
            gissue(i + 1, 1, gsem1)

        @pl.when(jnp.logical_and(i + 1 < NCHUNK, p == 1))
        def _():
            gissue(i + 1, 0, gsem0)

        @pl.when(jnp.logical_and(i >= 2, p == 0))
        def _():
            pltpu.make_async_copy(ev4.at[0], e_hbm.at[pl.ds(base, C)], esem0).wait()

        @pl.when(jnp.logical_and(i >= 2, p == 1))
        def _():
            pltpu.make_async_copy(ev4.at[1], e_hbm.at[pl.ds(base, C)], esem1).wait()

        def vb(j, carry2):
            rows = j * (16 // H) + row_off
            t = (plsc.load_gather(asv.at[p], [rows, col]) +
                 plsc.load_gather(adv.at[p], [rows, col]))
            t = jnp.where(t >= 0.0, t, 0.2 * t)
            plsc.store_scatter(ev4.at[p], [rows, col], t)
            plsc.store_scatter(ev16.at[p], [rows, col], t)
            return carry2

        lax.fori_loop(0, C * H // 16, vb, 0)

        @pl.when(p == 0)
        def _():
            pltpu.async_copy(ev4.at[0], e_hbm.at[pl.ds(base, C)], esem0)

        @pl.when(p == 1)
        def _():
            pltpu.async_copy(ev4.at[1], e_hbm.at[pl.ds(base, C)], esem1)

        pltpu.sync_copy(ev16.at[p], dsh.at[srcall.at[i]], add=True)
        return carry

    lax.fori_loop(0, NCHUNK, chunk, 0)
    pltpu.make_async_copy(ev4.at[0], e_hbm.at[pl.ds(base_w, C)], esem0).wait()
    pltpu.make_async_copy(ev4.at[1], e_hbm.at[pl.ds(base_w, C)], esem1).wait()
    plsc.subcore_barrier()
    pltpu.sync_copy(dsh.at[pl.ds(s * ROWS_T, ROWS_T)],
                    dp_hbm.at[c, pl.ds(s * ROWS_T, ROWS_T)])


def _rdenom_body(dp_ref, rd_ref):
    rd_ref[...] = 1.0 / (dp_ref[0] + dp_ref[1])


def _rdenom(dp):
    dpr = dp.reshape(NC, NP * HP // 128, 128)
    out = pl.pallas_call(
        _rdenom_body,
        out_shape=jax.ShapeDtypeStruct((NP * HP // 128, 128), _f32),
    )(dpr)
    return out.reshape(NP, HP)


@functools.partial(
    pl.kernel,
    compiler_params=_sc_params,
    out_type=jax.ShapeDtypeStruct((NC, NP, D), _f32),
    mesh=_mesh,
    scratch_types=[
        pltpu.VMEM((NCH2, C2), _i32),
        pltpu.VMEM((NCH2, C2), _i32),
        pltpu.VMEM((2, C2, H), _f32),
        pltpu.VMEM((2, C2, HP), _f32),
        pltpu.VMEM((C2,), _f32),
        pltpu.VMEM((2, C2, D), _f32),
        pltpu.VMEM_SHARED((NP, D), _f32),
        pltpu.SemaphoreType.DMA,
        pltpu.SemaphoreType.DMA,
    ],
)
def _pass2(src2_hbm, dst2_hbm, e_hbm, rd_hbm, fch_hbm, zerosD_hbm,
           op_hbm,
           srcall, dstall, ev, rdv, coeffv, grows, osh,
           lsem0, lsem1):
    c = lax.axis_index("c")
    s = lax.axis_index("s")
    w = c * NS + s
    base_w = w * PER_W
    pltpu.sync_copy(zerosD_hbm.at[pl.ds(s * ROWS_T, ROWS_T)],
                    osh.at[pl.ds(s * ROWS_T, ROWS_T)])
    pltpu.sync_copy(src2_hbm.at[pl.ds(w * NCH2, NCH2)], srcall)
    pltpu.sync_copy(dst2_hbm.at[pl.ds(w * NCH2, NCH2)], dstall)
    plsc.subcore_barrier()

    iota = lax.iota(_i32, 16)

    def lissue(i, p, sem):
        base = base_w + i * C2
        pltpu.async_copy(e_hbm.at[pl.ds(base, C2)], ev.at[p], sem)
        pltpu.async_copy(rd_hbm.at[srcall.at[i]], rdv.at[p], sem)
        pltpu.async_copy(fch_hbm.at[dstall.at[i]], grows.at[p], sem)

    def lwait(i, p, sem):
        base = base_w + i * C2
        pltpu.make_async_copy(e_hbm.at[pl.ds(base, C2)], ev.at[p], sem).wait()
        pltpu.make_async_copy(rd_hbm.at[srcall.at[i]], rdv.at[p], sem).wait()
        pltpu.make_async_copy(fch_hbm.at[dstall.at[i]], grows.at[p], sem).wait()

    lissue(0, 0, lsem0)

    def chunk(i, carry):
        p = lax.rem(i, 2)

        @pl.when(p == 0)
        def _():
            lwait(i, 0, lsem0)

        @pl.when(p == 1)
        def _():
            lwait(i, 1, lsem1)

        @pl.when(jnp.logical_and(i + 1 < NCH2, p == 0))
        def _():
            lissue(i + 1, 1, lsem1)

        @pl.when(jnp.logical_and(i + 1 < NCH2, p == 1))
        def _():
            lissue(i + 1, 0, lsem0)

        def cb(j, carry2):
            ridx = j * 16 + iota
            acc = jnp.zeros((16,), _f32)
            for h in range(H):
                hidx = jnp.full((16,), h, _i32)
                acc = acc + (plsc.load_gather(ev.at[p], [ridx, hidx]) *
                             plsc.load_gather(rdv.at[p], [ridx, hidx]))
            coeffv[pl.ds(j * 16, 16)] = acc * (1.0 / H)
            return carry2

        lax.fori_loop(0, C2 // 16, cb, 0)

        def sb(j, carry2):
            cbv = plsc.load_gather(coeffv, [jnp.full((16,), j, _i32)])
            for k in range(D // 16):
                grows[p, j, pl.ds(k * 16, 16)] = (
                    grows[p, j, pl.ds(k * 16, 16)] * cbv)
            return carry2

        lax.fori_loop(0, C2, sb, 0)

        pltpu.sync_copy(grows.at[p], osh.at[srcall.at[i]], add=True)
        return carry

    lax.fori_loop(0, NCH2, chunk, 0)
    plsc.subcore_barrier()
    pltpu.sync_copy(osh.at[pl.ds(s * ROWS_T, ROWS_T)],
                    op_hbm.at[c, pl.ds(s * ROWS_T, ROWS_T)])


def _final_body(op_ref, o_ref):
    o_ref[...] = jnp.maximum(op_ref[0] + op_ref[1], 0.0)


def _final(op):
    grid = (N // _RB,)
    return pl.pallas_call(
        _final_body,
        grid=grid,
        in_specs=[pl.BlockSpec((NC, _RB, D), lambda i: (0, i, 0))],
        out_specs=pl.BlockSpec((_RB, D), lambda i: (i, 0)),
        out_shape=jax.ShapeDtypeStruct((N, D), _f32),
    )(op)


def kernel(x, edge_index, attn_w, attn_b, fc_w, fc_b):
    src = edge_index[0].astype(_i32)
    dst = edge_index[1].astype(_i32)
    src2 = src.reshape(NROW, C)
    dst2 = dst.reshape(NROW, C)
    src2b = src.reshape(NROW2, C2)
    dst2b = dst.reshape(NROW2, C2)
    fc_wT = fc_w.T
    w1T = jnp.zeros((D, HP), _f32).at[:, :H].set(attn_w[:, :D].T)
    w2T = jnp.zeros((D, HP), _f32).at[:, :H].set(attn_w[:, D:].T)
    fc_b2 = fc_b.reshape(1, D)
    attn_b2 = jnp.zeros((1, HP), _f32).at[0, :H].set(attn_b)

    fc_h, a_src, a_dst = _dense(x, fc_wT, fc_b2, w1T, w2T, attn_b2)

    zeros16 = jnp.zeros((NP, HP), _f32)
    zerosD = jnp.zeros((NP, D), _f32)

    e, dp = _pass1(a_src, a_dst, src2, dst2, zeros16)
    rd = _rdenom(dp)
    op = _pass2(src2b, dst2b, e, rd, fc_h, zerosD)
    return _final(op[:, :N])


# speedup vs baseline: 10.9689x; 1.0534x over previous
import functools

import jax
import jax.numpy as jnp
from jax import lax
from jax.experimental import pallas as pl
from jax.experimental.pallas import tpu as pltpu
from jax.experimental.pallas import tpu_sc as plsc

N = 10000
E = 320000
D = 128
H = 4
HP = 16
NC = 2
NS = 16
NW = NC * NS
PER_W = E // NW
C = 80
NCHUNK = PER_W // C
NROW = E // C
C2 = 80
NCH2 = PER_W // C2
NROW2 = E // C2
NP = 10240
ROWS_T = NP // NS

_f32 = jnp.float32
_i32 = jnp.int32

_mesh = plsc.VectorSubcoreMesh(
    core_axis_name="c", subcore_axis_name="s", num_cores=NC, num_subcores=NS)

_sc_params = pltpu.CompilerParams(
    needs_layout_passes=False, use_tc_tiling_on_sc=False)


_RB = 400


def _dense_body(x_ref, fcw_ref, fcb_ref, w1_ref, w2_ref, ab_ref,
                fch_ref, as_ref, ad_ref):
    xb = x_ref[...]
    fch_ref[...] = jnp.dot(xb, fcw_ref[...],
                           preferred_element_type=_f32) + fcb_ref[...]
    as_ref[...] = jnp.dot(xb, w1_ref[...],
                          preferred_element_type=_f32) + ab_ref[...]
    ad_ref[...] = jnp.dot(xb, w2_ref[...],
                          preferred_element_type=_f32)


def _dense(x, fc_wT, fc_b2, w1T, w2T, attn_b2):
    grid = (N // _RB,)
    return pl.pallas_call(
        _dense_body,
        grid=grid,
        in_specs=[
            pl.BlockSpec((_RB, D), lambda i: (i, 0)),
            pl.BlockSpec((D, D), lambda i: (0, 0)),
            pl.BlockSpec((1, D), lambda i: (0, 0)),
            pl.BlockSpec((D, HP), lambda i: (0, 0)),
            pl.BlockSpec((D, HP), lambda i: (0, 0)),
            pl.BlockSpec((1, HP), lambda i: (0, 0)),
        ],
        out_specs=[
            pl.BlockSpec((_RB, D), lambda i: (i, 0)),
            pl.BlockSpec((_RB, HP), lambda i: (i, 0)),
            pl.BlockSpec((_RB, HP), lambda i: (i, 0)),
        ],
        out_shape=[
            jax.ShapeDtypeStruct((N, D), _f32),
            jax.ShapeDtypeStruct((N, HP), _f32),
            jax.ShapeDtypeStruct((N, HP), _f32),
        ],
    )(x, fc_wT, fc_b2, w1T, w2T, attn_b2)


@functools.partial(
    pl.kernel,
    compiler_params=_sc_params,
    out_type=(jax.ShapeDtypeStruct((E, H), _f32),
              jax.ShapeDtypeStruct((NC, NP, HP), _f32)),
    mesh=_mesh,
    scratch_types=[
        pltpu.VMEM((NCHUNK, C), _i32),
        pltpu.VMEM((NCHUNK, C), _i32),
        pltpu.VMEM((2, C, HP), _f32),
        pltpu.VMEM((2, C, HP), _f32),
        pltpu.VMEM((2, C, H), _f32),
        pltpu.VMEM((2, C, HP), _f32),
        pltpu.VMEM_SHARED((NP, HP), _f32),
        pltpu.SemaphoreType.DMA,
        pltpu.SemaphoreType.DMA,
        pltpu.SemaphoreType.DMA,
        pltpu.SemaphoreType.DMA,
    ],
)
def _pass1(a_src_hbm, a_dst_hbm, src2_hbm, dst2_hbm, zeros16_hbm,
           e_hbm, dp_hbm,
           srcall, dstall, asv, adv, ev4, ev16, dsh,
           gsem0, gsem1, esem0, esem1):
    c = lax.axis_index("c")
    s = lax.axis_index("s")
    w = c * NS + s
    base_w = w * PER_W
    pltpu.sync_copy(zeros16_hbm.at[pl.ds(s * ROWS_T, ROWS_T)],
                    dsh.at[pl.ds(s * ROWS_T, ROWS_T)])
    pltpu.sync_copy(src2_hbm.at[pl.ds(w * NCHUNK, NCHUNK)], srcall)
    pltpu.sync_copy(dst2_hbm.at[pl.ds(w * NCHUNK, NCHUNK)], dstall)

    zero16 = jnp.zeros((16,), _f32)

    def zb(j, carry):
        ev16[0, j, :] = zero16
        ev16[1, j, :] = zero16
        return carry

    lax.fori_loop(0, C, zb, 0)
    plsc.subcore_barrier()

    iota = lax.iota(_i32, 16)
    row_off = lax.shift_right_logical(iota, 2)
    col = lax.bitwise_and(iota, 3)

    def gissue(i, p, sem):
        pltpu.async_copy(a_src_hbm.at[srcall.at[i]], asv.at[p], sem)
        pltpu.async_copy(a_dst_hbm.at[dstall.at[i]], adv.at[p], sem)

    def gwait(i, p, sem):
        pltpu.make_async_copy(a_src_hbm.at[srcall.at[i]], asv.at[p], sem).wait()
        pltpu.make_async_copy(a_dst_hbm.at[dstall.at[i]], adv.at[p], sem).wait()

    gissue(0, 0, gsem0)

    def chunk(i, carry):
        p = lax.rem(i, 2)
        base = base_w + i * C

        @pl.when(p == 0)
        def _():
            gwait(i, 0, gsem0)

        @pl.when(p == 1)
        def _():
            gwait(i, 1, gsem1)

        @pl.when(jnp.logical_and(i + 1 < NCHUNK, p == 0))
        def _():
            gissue(i + 1, 1, gsem1)

        @pl.when(jnp.logical_and(i + 1 < NCHUNK, p == 1))
        def _():
            gissue(i + 1, 0, gsem0)

        @pl.when(jnp.logical_and(i >= 2, p == 0))
        def _():
            pltpu.make_async_copy(ev4.at[0], e_hbm.at[pl.ds(base, C)], esem0).wait()

        @pl.when(jnp.logical_and(i >= 2, p == 1))
        def _():
            pltpu.make_async_copy(ev4.at[1], e_hbm.at[pl.ds(base, C)], esem1).wait()

        @plsc.parallel_loop(0, C * H // 16, unroll=4)
        def vb(j):
            rows = j * (16 // H) + row_off
            t = (plsc.load_gather(asv.at[p], [rows, col]) +
                 plsc.load_gather(adv.at[p], [rows, col]))
            t = jnp.where(t >= 0.0, t, 0.2 * t)
            plsc.store_scatter(ev4.at[p], [rows, col], t)
            plsc.store_scatter(ev16.at[p], [rows, col], t)

        @pl.when(p == 0)
        def _():
            pltpu.async_copy(ev4.at[0], e_hbm.at[pl.ds(base, C)], esem0)

        @pl.when(p == 1)
        def _():
            pltpu.async_copy(ev4.at[1], e_hbm.at[pl.ds(base, C)], esem1)

        pltpu.sync_copy(ev16.at[p], dsh.at[srcall.at[i]], add=True)
        return carry

    lax.fori_loop(0, NCHUNK, chunk, 0)
    pltpu.make_async_copy(ev4.at[0], e_hbm.at[pl.ds(base_w, C)], esem0).wait()
    pltpu.make_async_copy(ev4.at[1], e_hbm.at[pl.ds(base_w, C)], esem1).wait()
    plsc.subcore_barrier()
    pltpu.sync_copy(dsh.at[pl.ds(s * ROWS_T, ROWS_T)],
                    dp_hbm.at[c, pl.ds(s * ROWS_T, ROWS_T)])


def _rdenom_body(dp_ref, rd_ref):
    rd_ref[...] = 1.0 / (dp_ref[0] + dp_ref[1])


def _rdenom(dp):
    dpr = dp.reshape(NC, NP * HP // 128, 128)
    out = pl.pallas_call(
        _rdenom_body,
        out_shape=jax.ShapeDtypeStruct((NP * HP // 128, 128), _f32),
    )(dpr)
    return out.reshape(NP, HP)


@functools.partial(
    pl.kernel,
    compiler_params=_sc_params,
    out_type=jax.ShapeDtypeStruct((NC, NP, D), _f32),
    mesh=_mesh,
    scratch_types=[
        pltpu.VMEM((NCH2, C2), _i32),
        pltpu.VMEM((NCH2, C2), _i32),
        pltpu.VMEM((2, C2, H), _f32),
        pltpu.VMEM((2, C2, HP), _f32),
        pltpu.VMEM((C2,), _f32),
        pltpu.VMEM((2, C2, D), _f32),
        pltpu.VMEM_SHARED((NP, D), _f32),
        pltpu.SemaphoreType.DMA,
        pltpu.SemaphoreType.DMA,
    ],
)
def _pass2(src2_hbm, dst2_hbm, e_hbm, rd_hbm, fch_hbm, zerosD_hbm,
           op_hbm,
           srcall, dstall, ev, rdv, coeffv, grows, osh,
           lsem0, lsem1):
    c = lax.axis_index("c")
    s = lax.axis_index("s")
    w = c * NS + s
    base_w = w * PER_W
    pltpu.sync_copy(zerosD_hbm.at[pl.ds(s * ROWS_T, ROWS_T)],
                    osh.at[pl.ds(s * ROWS_T, ROWS_T)])
    pltpu.sync_copy(src2_hbm.at[pl.ds(w * NCH2, NCH2)], srcall)
    pltpu.sync_copy(dst2_hbm.at[pl.ds(w * NCH2, NCH2)], dstall)
    plsc.subcore_barrier()

    iota = lax.iota(_i32, 16)

    def lissue(i, p, sem):
        base = base_w + i * C2
        pltpu.async_copy(e_hbm.at[pl.ds(base, C2)], ev.at[p], sem)
        pltpu.async_copy(rd_hbm.at[srcall.at[i]], rdv.at[p], sem)
        pltpu.async_copy(fch_hbm.at[dstall.at[i]], grows.at[p], sem)

    def lwait(i, p, sem):
        base = base_w + i * C2
        pltpu.make_async_copy(e_hbm.at[pl.ds(base, C2)], ev.at[p], sem).wait()
        pltpu.make_async_copy(rd_hbm.at[srcall.at[i]], rdv.at[p], sem).wait()
        pltpu.make_async_copy(fch_hbm.at[dstall.at[i]], grows.at[p], sem).wait()

    lissue(0, 0, lsem0)

    def chunk(i, carry):
        p = lax.rem(i, 2)

        @pl.when(p == 0)
        def _():
            lwait(i, 0, lsem0)

        @pl.when(p == 1)
        def _():
            lwait(i, 1, lsem1)

        @pl.when(jnp.logical_and(i + 1 < NCH2, p == 0))
        def _():
            lissue(i + 1, 1, lsem1)

        @pl.when(jnp.logical_and(i + 1 < NCH2, p == 1))
        def _():
            lissue(i + 1, 0, lsem0)

        def cb(j, carry2):
            ridx = j * 16 + iota
            acc = jnp.zeros((16,), _f32)
            for h in range(H):
                hidx = jnp.full((16,), h, _i32)
                acc = acc + (plsc.load_gather(ev.at[p], [ridx, hidx]) *
                             plsc.load_gather(rdv.at[p], [ridx, hidx]))
            coeffv[pl.ds(j * 16, 16)] = acc * (1.0 / H)
            return carry2

        lax.fori_loop(0, C2 // 16, cb, 0)

        @plsc.parallel_loop(0, C2, unroll=4)
        def sb(j):
            cbv = plsc.load_gather(coeffv, [jnp.full((16,), j, _i32)])
            for k in range(D // 16):
                grows[p, j, pl.ds(k * 16, 16)] = (
                    grows[p, j, pl.ds(k * 16, 16)] * cbv)

        pltpu.sync_copy(grows.at[p], osh.at[srcall.at[i]], add=True)
        return carry

    lax.fori_loop(0, NCH2, chunk, 0)
    plsc.subcore_barrier()
    pltpu.sync_copy(osh.at[pl.ds(s * ROWS_T, ROWS_T)],
                    op_hbm.at[c, pl.ds(s * ROWS_T, ROWS_T)])


def _final_body(op_ref, o_ref):
    o_ref[...] = jnp.maximum(op_ref[0] + op_ref[1], 0.0)


def _final(op):
    grid = (N // _RB,)
    return pl.pallas_call(
        _final_body,
        grid=grid,
        in_specs=[pl.BlockSpec((NC, _RB, D), lambda i: (0, i, 0))],
        out_specs=pl.BlockSpec((_RB, D), lambda i: (i, 0)),
        out_shape=jax.ShapeDtypeStruct((N, D), _f32),
    )(op)


def kernel(x, edge_index, attn_w, attn_b, fc_w, fc_b):
    src = edge_index[0].astype(_i32)
    dst = edge_index[1].astype(_i32)
    src2 = src.reshape(NROW, C)
    dst2 = dst.reshape(NROW, C)
    src2b = src.reshape(NROW2, C2)
    dst2b = dst.reshape(NROW2, C2)
    fc_wT = fc_w.T
    w1T = jnp.zeros((D, HP), _f32).at[:, :H].set(attn_w[:, :D].T)
    w2T = jnp.zeros((D, HP), _f32).at[:, :H].set(attn_w[:, D:].T)
    fc_b2 = fc_b.reshape(1, D)
    attn_b2 = jnp.zeros((1, HP), _f32).at[0, :H].set(attn_b)

    fc_h, a_src, a_dst = _dense(x, fc_wT, fc_b2, w1T, w2T, attn_b2)

    zeros16 = jnp.zeros((NP, HP), _f32)
    zerosD = jnp.zeros((NP, D), _f32)

    e, dp = _pass1(a_src, a_dst, src2, dst2, zeros16)
    rd = _rdenom(dp)
    op = _pass2(src2b, dst2b, e, rd, fc_h, zerosD)
    return _final(op[:, :N])
